# baseline (speedup 1.0000x reference)
import functools

import jax
import jax.numpy as jnp
from jax import lax
from jax.experimental import pallas as pl
from jax.experimental.pallas import tpu as pltpu
from jax.experimental.pallas import tpu_sc as plsc

BATCH = 16384
EMBED_DIM = 64
PACK_DIM = 128
LANES = 16
NUM_CORES = 2
NUM_SUBCORES = 16
NUM_WORKERS = NUM_CORES * NUM_SUBCORES
BPW = BATCH // NUM_WORKERS
CHUNK = 128
NCHUNKS = BPW // CHUNK
GROUPS = CHUNK // LANES

TR_BK = 32768
TR_HALF = TR_BK // 2
NUM_NODES = 1000000
TR_GRID = (NUM_NODES + TR_BK - 1) // TR_BK
PACK_ROWS = TR_GRID * TR_HALF


TR_SUB = 8192


def _tr_body(x_ref, out_ref):
    for c in range(TR_HALF // TR_SUB):
        xl = lax.transpose(x_ref[:, pl.ds(c * TR_SUB, TR_SUB)], (1, 0))
        xr = lax.transpose(
            x_ref[:, pl.ds(TR_HALF + c * TR_SUB, TR_SUB)], (1, 0))
        out_ref[pl.ds(c * TR_SUB, TR_SUB), :] = jnp.concatenate(
            [xl, xr], axis=1)


def _pack_table(emb_t):
    return pl.pallas_call(
        _tr_body,
        grid=(TR_GRID,),
        compiler_params=pltpu.CompilerParams(
            vmem_limit_bytes=128 * 1024 * 1024),
        in_specs=[pl.BlockSpec((EMBED_DIM, TR_BK), lambda g: (0, g))],
        out_specs=pl.BlockSpec((TR_HALF, PACK_DIM), lambda g: (g, 0)),
        out_shape=jax.ShapeDtypeStruct((PACK_ROWS, PACK_DIM), jnp.float32),
    )(emb_t)


TR_BK_LOG2 = TR_BK.bit_length() - 1
TR_HALF_LOG2 = TR_HALF.bit_length() - 1


def _packed_row(n):
    return lax.shift_left(
        lax.shift_right_logical(n, TR_BK_LOG2), TR_HALF_LOG2
    ) + (n & (TR_HALF - 1))


def _dot_body(i_hbm, j_hbm, emb_hbm, out_hbm,
              idx_i, idx_j, sup_i, sup_j,
              bi0, bi1, bj0, bj1, out_v,
              si0, si1, sj0, sj1):
    c = lax.axis_index("c")
    s = lax.axis_index("s")
    wid = s * NUM_CORES + c

    pltpu.sync_copy(i_hbm.at[wid], idx_i)
    pltpu.sync_copy(j_hbm.at[wid], idx_j)

    for k in range(NCHUNKS):
        for t in range(CHUNK // LANES):
            sl = pl.ds(t * LANES, LANES)
            sup_i[k, sl] = _packed_row(idx_i[k, sl])
            sup_j[k, sl] = _packed_row(idx_j[k, sl])

    bufs_i = (bi0, bi1)
    bufs_j = (bj0, bj1)
    sems_i = (si0, si1)
    sems_j = (sj0, sj1)

    def fire(k):
        b = k % 2
        return (pltpu.async_copy(emb_hbm.at[sup_i.at[k]], bufs_i[b], sems_i[b]),
                pltpu.async_copy(emb_hbm.at[sup_j.at[k]], bufs_j[b], sems_j[b]))

    lane = lax.broadcasted_iota(jnp.int32, (LANES,), 0)
    inflight = fire(0)

    for k in range(NCHUNKS):
        b = k % 2
        for cp in inflight:
            cp.wait()
        if k + 1 < NCHUNKS:
            inflight = fire(k + 1)
        bi, bj = bufs_i[b], bufs_j[b]

        def group(g, _):
            sl = pl.ds(g * LANES, LANES)
            hi = (lax.shift_right_logical(idx_i[k, sl], TR_HALF_LOG2) & 1) \
                * EMBED_DIM
            hj = (lax.shift_right_logical(idx_j[k, sl], TR_HALF_LOG2) & 1) \
                * EMBED_DIM
            out_vec = jnp.zeros((LANES,), jnp.float32)
            for t in range(LANES):
                r = g * LANES + t
                hb_i = hi[t]
                hb_j = hj[t]
                acc = jnp.zeros((LANES,), jnp.float32)
                for d in range(EMBED_DIM // LANES):
                    vi = bi[r, pl.ds(hb_i + d * LANES, LANES)]
                    vj = bj[r, pl.ds(hb_j + d * LANES, LANES)]
                    acc = acc + vi * vj
                dot = jnp.sum(acc)
                out_vec = jnp.where(lane == t, dot, out_vec)
            out_v[pl.ds(k * CHUNK + g * LANES, LANES)] = out_vec
            return 0

        lax.fori_loop(0, GROUPS, group, 0)

    pltpu.sync_copy(out_v, out_hbm.at[pl.ds(wid * BPW, BPW)])


@jax.jit
def _sc_dot(i, j, node_emb):
    mesh = plsc.VectorSubcoreMesh(core_axis_name="c", subcore_axis_name="s")
    kfn = pl.kernel(
        _dot_body,
        mesh=mesh,
        compiler_params=pltpu.CompilerParams(
            needs_layout_passes=False, use_tc_tiling_on_sc=True),
        out_type=jax.ShapeDtypeStruct((BATCH,), jnp.float32),
        scratch_types=[
            pltpu.VMEM((NCHUNKS, CHUNK), jnp.int32),
            pltpu.VMEM((NCHUNKS, CHUNK), jnp.int32),
            pltpu.VMEM((NCHUNKS, CHUNK), jnp.int32),
            pltpu.VMEM((NCHUNKS, CHUNK), jnp.int32),
            pltpu.VMEM((CHUNK, PACK_DIM), jnp.float32),
            pltpu.VMEM((CHUNK, PACK_DIM), jnp.float32),
            pltpu.VMEM((CHUNK, PACK_DIM), jnp.float32),
            pltpu.VMEM((CHUNK, PACK_DIM), jnp.float32),
            pltpu.VMEM((BPW,), jnp.float32),
            pltpu.SemaphoreType.DMA,
            pltpu.SemaphoreType.DMA,
            pltpu.SemaphoreType.DMA,
            pltpu.SemaphoreType.DMA,
        ],
    )
    packed = _pack_table(node_emb.T)
    return kfn(i.reshape(NUM_WORKERS, NCHUNKS, CHUNK),
               j.reshape(NUM_WORKERS, NCHUNKS, CHUNK),
               packed)


def kernel(i, j, node_emb):
    return _sc_dot(i.astype(jnp.int32), j.astype(jnp.int32), node_emb)
